# baseline (speedup 1.0000x reference)
import jax
import jax.numpy as jnp
from jax import lax
from jax.experimental import pallas as pl
from jax.experimental.pallas import tpu as pltpu


_B, _C, _H, _W = 8, 1024, 24, 24
_HW = _H * _W
_ROWS = _B * _HW
_NT = _C // 128
_CH = 576
_NCH = _ROWS // _CH
_BIG = 1 << 20


def _argmax_rows(x):
    m = x[:, 0:128]
    tidx = jnp.zeros((_CH, 128), jnp.int32)
    for t in range(1, _NT):
        xt = x[:, 128 * t:128 * (t + 1)]
        gt = xt > m
        m = jnp.where(gt, xt, m)
        tidx = jnp.where(gt, t, tidx)
    rowmax = jnp.max(m, axis=1, keepdims=True)
    lane = lax.broadcasted_iota(jnp.int32, (_CH, 128), 1)
    cand = jnp.where(m == rowmax, 128 * tidx + lane, _BIG)
    return jnp.min(cand, axis=1)


def _body(x_hbm, xo_hbm, idx_ref, buf, sin, sout):
    def in_copy(i):
        return pltpu.make_async_copy(
            x_hbm.at[pl.ds(i * _CH, _CH)], buf.at[pl.ds(i * _CH, _CH)],
            sin.at[i])

    def out_copy(i):
        return pltpu.make_async_copy(
            buf.at[pl.ds(i * _CH, _CH)], xo_hbm.at[pl.ds(i * _CH, _CH)],
            sout.at[i])

    for i in range(_NCH):
        in_copy(i).start()
    for i in range(_NCH):
        in_copy(i).wait()
        out_copy(i).start()
        idx_ref[i, 0] = _argmax_rows(buf[pl.ds(i * _CH, _CH), :])
    for i in range(_NCH):
        out_copy(i).wait()


def kernel(inputs):
    xt = inputs.transpose(0, 2, 3, 1).reshape(_ROWS, _C)
    x_out, idx = pl.pallas_call(
        _body,
        in_specs=[pl.BlockSpec(memory_space=pl.ANY)],
        out_specs=[
            pl.BlockSpec(memory_space=pl.ANY),
            pl.BlockSpec(memory_space=pltpu.VMEM),
        ],
        out_shape=[
            jax.ShapeDtypeStruct((_ROWS, _C), jnp.float32),
            jax.ShapeDtypeStruct((_NCH, 1, _CH), jnp.int32),
        ],
        scratch_shapes=[
            pltpu.VMEM((_ROWS, _C), jnp.float32),
            pltpu.SemaphoreType.DMA((_NCH,)),
            pltpu.SemaphoreType.DMA((_NCH,)),
        ],
    )(xt)
    x_out = x_out.reshape(_B, _H, _W, _C).transpose(0, 3, 1, 2)
    return (x_out, idx.reshape(_B, 1, _H, _W))
